# baseline (speedup 1.0000x reference)
import functools
import numpy as np
import jax
import jax.numpy as jnp
from jax import lax
from jax.experimental import pallas as pl
from jax.experimental.pallas import tpu as pltpu
from jax.experimental.pallas import tpu_sc as plsc

D_MODEL = 1024
NUM_SETS = 64
K_TOP = 8
NEG = -1e30
BLK = 512

_SC_NUM_CORES = 2
_SC_NUM_SUBCORES = 16
_NW = _SC_NUM_CORES * _SC_NUM_SUBCORES


def _scores_body(x_ref, desc_ref, wq_ref, bq_ref, s_ref):
    scale = 1.0 / np.sqrt(D_MODEL)
    q = jax.lax.dot_general(
        x_ref[0], wq_ref[...], (((1,), (1,)), ((), ())),
        preferred_element_type=jnp.float32) + bq_ref[...]
    s_ref[0] = jax.lax.dot_general(
        q, desc_ref[0], (((1,), (1,)), ((), ())),
        preferred_element_type=jnp.float32) * scale


def _combine_body(sk_ref, sv_ref, set_ref, out_ref):
    sk = sk_ref[...]
    sv = sv_ref[...]
    lane = lax.broadcasted_iota(jnp.int32, sk.shape, 1)
    top8 = lane < K_TOP
    s8 = jnp.where(top8, sk, NEG)
    m = jnp.max(s8, axis=1, keepdims=True)
    e = jnp.where(top8 & (sk > NEG * 0.5), jnp.exp(s8 - m), 0.0)
    w = e / jnp.sum(e, axis=1, keepdims=True)
    sets = lax.broadcasted_iota(jnp.int32, (sk.shape[0], NUM_SETS), 1)
    dense = jnp.zeros((sk.shape[0], NUM_SETS), jnp.float32)
    for j in range(K_TOP):
        dense = dense + jnp.where(sv[:, j:j + 1] == sets,
                                  w[:, j:j + 1], 0.0)
    out_ref[0] = jax.lax.dot_general(
        dense, set_ref[0], (((1,), (0,)), ((), ())),
        preferred_element_type=jnp.float32)


_NCHUNK = 4


def _route_body(s_hbm, tts_hbm, sk_hbm, sv_hbm, s_v, tts_v, sk_v, sv_v,
                win_v, sem_t, sem_i, sem_o, per_w):
    wid = lax.axis_index("s") * _SC_NUM_CORES + lax.axis_index("c")
    base = wid * per_w
    ch = per_w // _NCHUNK

    h_t = pltpu.async_copy(tts_hbm.at[pl.ds(base * 16, per_w * 16)],
                           tts_v, sem_t)
    h_in = [
        pltpu.async_copy(
            s_hbm.at[pl.ds((base + c * ch) * NUM_SETS, ch * NUM_SETS)],
            s_v.at[pl.ds(c * ch * NUM_SETS, ch * NUM_SETS)], sem_i)
        for c in range(_NCHUNK)
    ]

    lanes = lax.iota(jnp.int32, 16)

    def token(t, _):
        idx = plsc.load_gather(tts_v, [t * 16 + lanes])
        plsc.store_scatter(win_v, [idx], lanes)
        keep = plsc.load_gather(win_v, [idx]) == lanes
        s = plsc.load_gather(s_v, [t * NUM_SETS + idx])
        s_m = jnp.where(keep, s, NEG)
        sk, sv = plsc.sort_key_val(s_m, idx, descending=True)
        sk_v[pl.ds(t * 16, 16)] = sk
        sv_v[pl.ds(t * 16, 16)] = sv
        return 0

    h_t.wait()
    h_out = []
    for c in range(_NCHUNK):
        h_in[c].wait()
        lax.fori_loop(c * ch, (c + 1) * ch, token, 0, unroll=4)
        h_out.append(pltpu.async_copy(
            sk_v.at[pl.ds(c * ch * 16, ch * 16)],
            sk_hbm.at[pl.ds((base + c * ch) * 16, ch * 16)], sem_o))
        h_out.append(pltpu.async_copy(
            sv_v.at[pl.ds(c * ch * 16, ch * 16)],
            sv_hbm.at[pl.ds((base + c * ch) * 16, ch * 16)], sem_o))
    for h in h_out:
        h.wait()


@jax.jit
def _run(token_states, set_states, desc_router, tts2, W_q, b_q2):
    batch, seq_len, d = token_states.shape
    nb = seq_len // BLK
    tokens = batch * seq_len
    per_w = tokens // _NW

    scores = pl.pallas_call(
        _scores_body,
        grid=(batch, nb),
        in_specs=[
            pl.BlockSpec((1, BLK, d), lambda b, i: (b, i, 0)),
            pl.BlockSpec((1, NUM_SETS, d), lambda b, i: (b, 0, 0)),
            pl.BlockSpec((d, d), lambda b, i: (0, 0)),
            pl.BlockSpec((1, d), lambda b, i: (0, 0)),
        ],
        out_specs=pl.BlockSpec((1, BLK, NUM_SETS), lambda b, i: (b, i, 0)),
        out_shape=jax.ShapeDtypeStruct((batch, seq_len, NUM_SETS),
                                       jnp.float32),
    )(token_states, desc_router, W_q, b_q2)
    scores_f = scores.reshape(tokens * NUM_SETS)

    mesh = plsc.VectorSubcoreMesh(core_axis_name="c", subcore_axis_name="s",
                                  num_cores=_SC_NUM_CORES,
                                  num_subcores=_SC_NUM_SUBCORES)
    sk_f, sv_f = pl.kernel(
        functools.partial(_route_body, per_w=per_w),
        out_type=[
            jax.ShapeDtypeStruct((tokens * 16,), jnp.float32),
            jax.ShapeDtypeStruct((tokens * 16,), jnp.int32),
        ],
        mesh=mesh,
        compiler_params=pltpu.CompilerParams(needs_layout_passes=False),
        scratch_types=[
            pltpu.VMEM((per_w * NUM_SETS,), jnp.float32),
            pltpu.VMEM((per_w * 16,), jnp.int32),
            pltpu.VMEM((per_w * 16,), jnp.float32),
            pltpu.VMEM((per_w * 16,), jnp.int32),
            pltpu.VMEM((NUM_SETS,), jnp.int32),
            pltpu.SemaphoreType.DMA,
            pltpu.SemaphoreType.DMA,
            pltpu.SemaphoreType.DMA,
        ],
    )(scores_f, tts2.reshape(-1))
    sk = sk_f.reshape(tokens, 16)
    sv = sv_f.reshape(tokens, 16)

    out = pl.pallas_call(
        _combine_body,
        grid=(batch, nb),
        in_specs=[
            pl.BlockSpec((BLK, 16), lambda b, i, nb=nb: (b * nb + i, 0)),
            pl.BlockSpec((BLK, 16), lambda b, i, nb=nb: (b * nb + i, 0)),
            pl.BlockSpec((1, NUM_SETS, d), lambda b, i: (b, 0, 0)),
        ],
        out_specs=pl.BlockSpec((1, BLK, d), lambda b, i: (b, i, 0)),
        out_shape=jax.ShapeDtypeStruct((batch, seq_len, d), jnp.float32),
    )(sk, sv, set_states)
    return out


def kernel(token_states, set_states, desc_router, token_to_sets, W_q, b_q):
    batch = token_states.shape[0]
    tts = token_to_sets.astype(jnp.int32)
    tts2 = jnp.concatenate([tts] * batch, axis=0)
    return _run(token_states, set_states, desc_router, tts2, W_q,
                b_q.reshape(1, -1))


# speedup vs baseline: 1.0639x; 1.0639x over previous
import functools
import numpy as np
import jax
import jax.numpy as jnp
from jax import lax
from jax.experimental import pallas as pl
from jax.experimental.pallas import tpu as pltpu
from jax.experimental.pallas import tpu_sc as plsc

D_MODEL = 1024
NUM_SETS = 64
K_TOP = 8
NEG = -1e30
BLK = 512

_SC_NUM_CORES = 2
_SC_NUM_SUBCORES = 16
_NW = _SC_NUM_CORES * _SC_NUM_SUBCORES


def _scores_body(x_ref, desc_ref, wq_ref, bq_ref, s_ref):
    scale = 1.0 / np.sqrt(D_MODEL)
    q = jax.lax.dot_general(
        x_ref[0], wq_ref[...], (((1,), (1,)), ((), ())),
        preferred_element_type=jnp.float32) + bq_ref[...]
    s_ref[0] = jax.lax.dot_general(
        q, desc_ref[0], (((1,), (1,)), ((), ())),
        preferred_element_type=jnp.float32) * scale


def _combine_body(w_ref, set_ref, out_ref):
    out_ref[0] = jax.lax.dot_general(
        w_ref[...], set_ref[0], (((1,), (0,)), ((), ())),
        preferred_element_type=jnp.float32)


_NCHUNK = 4


def _route_body(s_hbm, tts_hbm, w_hbm, s_v, tts_v, wout_v, win_v,
                sem_t, sem_i, sem_o, per_w):
    wid = lax.axis_index("s") * _SC_NUM_CORES + lax.axis_index("c")
    base = wid * per_w
    ch = per_w // _NCHUNK

    h_t = pltpu.async_copy(tts_hbm.at[pl.ds(base * 16, per_w * 16)],
                           tts_v, sem_t)
    h_in = [
        pltpu.async_copy(
            s_hbm.at[pl.ds((base + c * ch) * NUM_SETS, ch * NUM_SETS)],
            s_v.at[pl.ds(c * ch * NUM_SETS, ch * NUM_SETS)], sem_i)
        for c in range(_NCHUNK)
    ]

    lanes = lax.iota(jnp.int32, 16)
    zeros = jnp.zeros((16,), jnp.float32)

    def token(t, _):
        idx = plsc.load_gather(tts_v, [t * 16 + lanes])
        plsc.store_scatter(win_v, [idx], lanes)
        keep = plsc.load_gather(win_v, [idx]) == lanes
        s = plsc.load_gather(s_v, [t * NUM_SETS + idx])
        s_m = jnp.where(keep, s, NEG)
        sk, sv = plsc.sort_key_val(s_m, idx, descending=True)
        top8 = lanes < K_TOP
        s8 = jnp.where(top8, sk, NEG)
        m = jnp.max(s8)
        e = jnp.exp(s8 - m)
        w = e / jnp.sum(e)
        valid = top8 & (sk > NEG * 0.5)
        for jj in range(4):
            wout_v[pl.ds(t * NUM_SETS + 16 * jj, 16)] = zeros
        plsc.store_scatter(wout_v, [t * NUM_SETS + sv], w, mask=valid)
        return 0

    h_t.wait()
    h_out = []
    for c in range(_NCHUNK):
        h_in[c].wait()
        lax.fori_loop(c * ch, (c + 1) * ch, token, 0, unroll=4)
        h_out.append(pltpu.async_copy(
            wout_v.at[pl.ds(c * ch * NUM_SETS, ch * NUM_SETS)],
            w_hbm.at[pl.ds((base + c * ch) * NUM_SETS, ch * NUM_SETS)],
            sem_o))
    for h in h_out:
        h.wait()


@jax.jit
def _run(token_states, set_states, desc_router, tts2, W_q, b_q2):
    batch, seq_len, d = token_states.shape
    nb = seq_len // BLK
    tokens = batch * seq_len
    per_w = tokens // _NW

    scores = pl.pallas_call(
        _scores_body,
        grid=(batch, nb),
        in_specs=[
            pl.BlockSpec((1, BLK, d), lambda b, i: (b, i, 0)),
            pl.BlockSpec((1, NUM_SETS, d), lambda b, i: (b, 0, 0)),
            pl.BlockSpec((d, d), lambda b, i: (0, 0)),
            pl.BlockSpec((1, d), lambda b, i: (0, 0)),
        ],
        out_specs=pl.BlockSpec((1, BLK, NUM_SETS), lambda b, i: (b, i, 0)),
        out_shape=jax.ShapeDtypeStruct((batch, seq_len, NUM_SETS),
                                       jnp.float32),
    )(token_states, desc_router, W_q, b_q2)
    scores_f = scores.reshape(tokens * NUM_SETS)

    mesh = plsc.VectorSubcoreMesh(core_axis_name="c", subcore_axis_name="s",
                                  num_cores=_SC_NUM_CORES,
                                  num_subcores=_SC_NUM_SUBCORES)
    weights = pl.kernel(
        functools.partial(_route_body, per_w=per_w),
        out_type=jax.ShapeDtypeStruct((tokens * NUM_SETS,), jnp.float32),
        mesh=mesh,
        compiler_params=pltpu.CompilerParams(needs_layout_passes=False),
        scratch_types=[
            pltpu.VMEM((per_w * NUM_SETS,), jnp.float32),
            pltpu.VMEM((per_w * 16,), jnp.int32),
            pltpu.VMEM((per_w * NUM_SETS,), jnp.float32),
            pltpu.VMEM((NUM_SETS,), jnp.int32),
            pltpu.SemaphoreType.DMA,
            pltpu.SemaphoreType.DMA,
            pltpu.SemaphoreType.DMA,
        ],
    )(scores_f, tts2.reshape(-1))
    weights = weights.reshape(tokens, NUM_SETS)

    out = pl.pallas_call(
        _combine_body,
        grid=(batch, nb),
        in_specs=[
            pl.BlockSpec((BLK, NUM_SETS),
                         lambda b, i, nb=nb: (b * nb + i, 0)),
            pl.BlockSpec((1, NUM_SETS, d), lambda b, i: (b, 0, 0)),
        ],
        out_specs=pl.BlockSpec((1, BLK, d), lambda b, i: (b, i, 0)),
        out_shape=jax.ShapeDtypeStruct((batch, seq_len, d), jnp.float32),
    )(weights, set_states)
    return out


def kernel(token_states, set_states, desc_router, token_to_sets, W_q, b_q):
    batch = token_states.shape[0]
    tts = token_to_sets.astype(jnp.int32)
    tts2 = jnp.concatenate([tts] * batch, axis=0)
    return _run(token_states, set_states, desc_router, tts2, W_q,
                b_q.reshape(1, -1))
